# baseline (speedup 1.0000x reference)
import functools

import jax
import jax.numpy as jnp
from jax import lax
from jax.experimental import pallas as pl
from jax.experimental.pallas import tpu as pltpu
from jax.experimental.pallas import tpu_sc as plsc

_B = 2048
_IN = 512
_OUT = 32
_T = 32
_DEPTH = 6
_S = 31
_L = 32
_F = 16
_TL = _T * _L
_K = _TL * _F

_BT = 256
_NLC = 128
_CK = _NLC * _F
_NCHUNK = _TL // _NLC

_NW = 32
_KPW = _K // _NW
_RC = 32
_NCH_SC = _KPW // _RC


def _smooth_step(t):
    tc = jnp.clip(t, -0.5, 0.5)
    return tc * (1.5 - 2.0 * tc * tc) + 0.5


def _sc_gather_body(xT_hbm, ids_hbm, out_hbm,
                    idx_v, rows0, rows1, gs0, gs1, ss0, ss1):
    wid = lax.axis_index("s") * 2 + lax.axis_index("c")
    base = wid * _KPW
    pltpu.sync_copy(ids_hbm.at[pl.ds(base, _KPW)], idx_v)
    bufs = (rows0, rows1)
    gsems = (gs0, gs1)
    ssems = (ss0, ss1)

    def _gather(j):
        b = bufs[j % 2]
        return pltpu.async_copy(
            xT_hbm.at[idx_v.at[pl.ds(j * _RC, _RC)]], b, gsems[j % 2])

    gd = [_gather(0), _gather(1)]
    for j in range(_NCH_SC):
        b = j % 2
        gd[b].wait()
        sd = pltpu.async_copy(
            bufs[b], out_hbm.at[pl.ds(base + j * _RC, _RC)], ssems[b])
        sd.wait()
        if j + 2 < _NCH_SC:
            gd[b] = _gather(j + 2)


def _sc_gather(xT, ids_flat):
    mesh = plsc.VectorSubcoreMesh(core_axis_name="c", subcore_axis_name="s")
    return pl.kernel(
        _sc_gather_body,
        mesh=mesh,
        out_type=jax.ShapeDtypeStruct((_K, _B // 2), jnp.uint32),
        scratch_types=[
            pltpu.VMEM((_KPW,), jnp.int32),
            pltpu.VMEM((_RC, _B // 2), jnp.uint32),
            pltpu.VMEM((_RC, _B // 2), jnp.uint32),
            pltpu.SemaphoreType.DMA,
            pltpu.SemaphoreType.DMA,
            pltpu.SemaphoreType.DMA,
            pltpu.SemaphoreType.DMA,
        ],
    )(xT, ids_flat)


def _route_kernel(xT_ref, slopes_ref, bias_ref, aT_ref):
    t = jax.lax.dot_general(
        slopes_ref[...], xT_ref[...], (((1,), (0,)), ((), ())),
        preferred_element_type=jnp.float32)
    s = _smooth_step(t + bias_ref[...])
    s3 = s.reshape(_T, _S, _BT)
    aT = None
    for d in range(_DEPTH - 1):
        nb, ne = 2 ** d - 1, 2 ** (d + 1) - 1
        lvl = s3[:, nb:ne, :].reshape(_T, ne - nb, 1, _BT)
        rep = jnp.broadcast_to(lvl, (_T, ne - nb, _L // (ne - nb), _BT))
        rep = rep.reshape(_T, _L, _BT)
        lidx = jax.lax.broadcasted_iota(jnp.int32, (1, _L, 1), 1)
        bit = ((lidx >> (_DEPTH - 2 - d)) & 1).astype(jnp.float32)
        f = (2.0 * bit - 1.0) * rep + (1.0 - bit)
        aT = f if aT is None else aT * f
    aT_ref[...] = aT.reshape(_TL, _BT)


def _main_kernel(xf_ref, ac_ref, afull_ref, w2f_ref, wb_ref, out_ref):
    c = pl.program_id(1)

    @pl.when(c == 0)
    def _init():
        out_ref[...] = jax.lax.dot_general(
            afull_ref[...], wb_ref[...], (((0,), (0,)), ((), ())),
            preferred_element_type=jnp.float32)

    a_c = ac_ref[...].reshape(_NLC, 1, _BT)
    a_rep = jnp.broadcast_to(a_c, (_NLC, _F, _BT)).reshape(_CK, _BT)
    y = a_rep * xf_ref[...].astype(jnp.float32)
    out_ref[...] += jax.lax.dot_general(
        y, w2f_ref[...], (((0,), (0,)), ((), ())),
        preferred_element_type=jnp.float32)


@jax.jit
def kernel(x, split_coefs, leaves_feat_ids, leaves_coefs):
    xT = x.T
    slopes = split_coefs[:, :, :-1].reshape(_T * _S, _IN)
    bias = split_coefs[:, :, -1].reshape(_T * _S, 1)
    ids_flat = leaves_feat_ids.astype(jnp.int32).reshape(_K)
    w2f = jnp.transpose(leaves_coefs[:, :, :, :_F], (0, 1, 3, 2))
    w2f = w2f.reshape(_K, _OUT)
    wb = leaves_coefs[:, :, :, _F].reshape(_TL, _OUT)

    xu = jax.lax.bitcast_convert_type(
        xT.astype(jnp.bfloat16).reshape(_IN, _B // 2, 2), jnp.uint32)
    xfT = jax.lax.bitcast_convert_type(
        _sc_gather(xu, ids_flat), jnp.bfloat16).reshape(_K, _B)

    aT = pl.pallas_call(
        _route_kernel,
        grid=(_B // _BT,),
        in_specs=[
            pl.BlockSpec((_IN, _BT), lambda i: (0, i)),
            pl.BlockSpec((_T * _S, _IN), lambda i: (0, 0)),
            pl.BlockSpec((_T * _S, 1), lambda i: (0, 0)),
        ],
        out_specs=pl.BlockSpec((_TL, _BT), lambda i: (0, i)),
        out_shape=jax.ShapeDtypeStruct((_TL, _B), jnp.float32),
    )(xT, slopes, bias)

    return pl.pallas_call(
        _main_kernel,
        grid=(_B // _BT, _NCHUNK),
        in_specs=[
            pl.BlockSpec((_CK, _BT), lambda i, c: (c, i)),
            pl.BlockSpec((_NLC, _BT), lambda i, c: (c, i)),
            pl.BlockSpec((_TL, _BT), lambda i, c: (0, i)),
            pl.BlockSpec((_CK, _OUT), lambda i, c: (c, 0)),
            pl.BlockSpec((_TL, _OUT), lambda i, c: (0, 0)),
        ],
        out_specs=pl.BlockSpec((_BT, _OUT), lambda i, c: (i, 0)),
        out_shape=jax.ShapeDtypeStruct((_B, _OUT), jnp.float32),
    )(xfT, aT, aT, w2f, wb)


# speedup vs baseline: 3.0966x; 3.0966x over previous
import functools

import jax
import jax.numpy as jnp
from jax import lax
from jax.experimental import pallas as pl
from jax.experimental.pallas import tpu as pltpu
from jax.experimental.pallas import tpu_sc as plsc

_B = 2048
_IN = 512
_OUT = 32
_T = 32
_DEPTH = 6
_S = 31
_L = 32
_F = 16
_TL = _T * _L
_K = _TL * _F

_BT = 256
_BTH = 128
_NLC = 128
_CK = _NLC * _F
_NCHUNK = _TL // _NLC

_NW = 32
_KPW = _K // _NW
_RC = 32
_NCH_SC = _KPW // _RC


def _smooth_step(t):
    tc = jnp.clip(t, -0.5, 0.5)
    return tc * (1.5 - 2.0 * tc * tc) + 0.5


def _sc_gather_body(xT_hbm, ids_hbm, out_hbm,
                    idx_v, rows0, rows1, gs0, gs1, ss0, ss1):
    wid = lax.axis_index("s") * 2 + lax.axis_index("c")
    base = wid * _KPW
    pltpu.sync_copy(ids_hbm.at[pl.ds(base, _KPW)], idx_v)
    bufs = (rows0, rows1)
    gsems = (gs0, gs1)
    ssems = (ss0, ss1)

    def _gather(j):
        b = bufs[j % 2]
        return pltpu.async_copy(
            xT_hbm.at[idx_v.at[pl.ds(j * _RC, _RC)]], b, gsems[j % 2])

    gd = [_gather(0), _gather(1)]
    for j in range(_NCH_SC):
        b = j % 2
        gd[b].wait()
        sd = pltpu.async_copy(
            bufs[b], out_hbm.at[pl.ds(base + j * _RC, _RC)], ssems[b])
        sd.wait()
        if j + 2 < _NCH_SC:
            gd[b] = _gather(j + 2)


def _sc_gather(xT, ids_flat):
    mesh = plsc.VectorSubcoreMesh(core_axis_name="c", subcore_axis_name="s")
    return pl.kernel(
        _sc_gather_body,
        mesh=mesh,
        out_type=jax.ShapeDtypeStruct((_K, _B // 2), jnp.uint32),
        scratch_types=[
            pltpu.VMEM((_KPW,), jnp.int32),
            pltpu.VMEM((_RC, _B // 2), jnp.uint32),
            pltpu.VMEM((_RC, _B // 2), jnp.uint32),
            pltpu.SemaphoreType.DMA,
            pltpu.SemaphoreType.DMA,
            pltpu.SemaphoreType.DMA,
            pltpu.SemaphoreType.DMA,
        ],
    )(xT, ids_flat)


def _route_kernel(xT_ref, slopes_ref, bias_ref, aT_ref):
    t = jax.lax.dot_general(
        slopes_ref[...], xT_ref[...], (((1,), (0,)), ((), ())),
        preferred_element_type=jnp.float32)
    s = _smooth_step(t + bias_ref[...])
    s3 = s.reshape(_T, _S, _BT)
    aT = None
    for d in range(_DEPTH - 1):
        nb, ne = 2 ** d - 1, 2 ** (d + 1) - 1
        lvl = s3[:, nb:ne, :].reshape(_T, ne - nb, 1, _BT)
        rep = jnp.broadcast_to(lvl, (_T, ne - nb, _L // (ne - nb), _BT))
        rep = rep.reshape(_T, _L, _BT)
        lidx = jax.lax.broadcasted_iota(jnp.int32, (1, _L, 1), 1)
        bit = ((lidx >> (_DEPTH - 2 - d)) & 1).astype(jnp.float32)
        f = (2.0 * bit - 1.0) * rep + (1.0 - bit)
        aT = f if aT is None else aT * f
    aT_ref[...] = aT.reshape(_TL, _BT)


def _main_kernel(xf_ref, ac_ref, afull_ref, w2f_ref, wb_ref, out_ref):
    c = pl.program_id(1)
    w2f = w2f_ref[...]

    @pl.when(c == 0)
    def _init():
        af = afull_ref[...]
        for h in range(2):
            out_ref[h] = jax.lax.dot_general(
                af[:, h, :], wb_ref[...], (((0,), (0,)), ((), ())),
                preferred_element_type=jnp.float32)

    w = xf_ref[...]
    xs = (jax.lax.bitcast_convert_type(
              jax.lax.shift_left(w, jnp.uint32(16)), jnp.float32),
          jax.lax.bitcast_convert_type(
              w & jnp.uint32(0xFFFF0000), jnp.float32))
    a3 = ac_ref[...]
    for h in range(2):
        a_h = a3[:, h, :].reshape(_NLC, 1, _BTH)
        a_rep = jnp.broadcast_to(a_h, (_NLC, _F, _BTH)).reshape(_CK, _BTH)
        y = a_rep * xs[h]
        out_ref[h] += jax.lax.dot_general(
            y, w2f, (((0,), (0,)), ((), ())),
            preferred_element_type=jnp.float32)


@jax.jit
def kernel(x, split_coefs, leaves_feat_ids, leaves_coefs):
    xT = x.T
    slopes = split_coefs[:, :, :-1].reshape(_T * _S, _IN)
    bias = split_coefs[:, :, -1].reshape(_T * _S, 1)
    ids_flat = leaves_feat_ids.astype(jnp.int32).reshape(_K)
    w2f = jnp.transpose(leaves_coefs[:, :, :, :_F], (0, 1, 3, 2))
    w2f = w2f.reshape(_K, _OUT)
    wb = leaves_coefs[:, :, :, _F].reshape(_TL, _OUT)

    xu = jax.lax.bitcast_convert_type(
        jnp.transpose(xT.astype(jnp.bfloat16).reshape(_IN, 2, _B // 2),
                      (0, 2, 1)), jnp.uint32)
    xfT = _sc_gather(xu, ids_flat)

    aT = pl.pallas_call(
        _route_kernel,
        grid=(_B // _BT,),
        in_specs=[
            pl.BlockSpec((_IN, _BT), lambda i: (0, i)),
            pl.BlockSpec((_T * _S, _IN), lambda i: (0, 0)),
            pl.BlockSpec((_T * _S, 1), lambda i: (0, 0)),
        ],
        out_specs=pl.BlockSpec((_TL, _BT), lambda i: (0, i)),
        out_shape=jax.ShapeDtypeStruct((_TL, _B), jnp.float32),
    )(xT, slopes, bias)

    aT3 = aT.reshape(_TL, 2, _B // 2)
    out = pl.pallas_call(
        _main_kernel,
        grid=(_B // 2 // _BTH, _NCHUNK),
        in_specs=[
            pl.BlockSpec((_CK, _BTH), lambda i, c: (c, i)),
            pl.BlockSpec((_NLC, 2, _BTH), lambda i, c: (c, 0, i)),
            pl.BlockSpec((_TL, 2, _BTH), lambda i, c: (0, 0, i)),
            pl.BlockSpec((_CK, _OUT), lambda i, c: (c, 0)),
            pl.BlockSpec((_TL, _OUT), lambda i, c: (0, 0)),
        ],
        out_specs=pl.BlockSpec((2, _BTH, _OUT), lambda i, c: (0, i, 0)),
        out_shape=jax.ShapeDtypeStruct((2, _B // 2, _OUT), jnp.float32),
    )(xfT, aT3, aT3, w2f, wb)
    return out.reshape(_B, _OUT)
